# Initial kernel scaffold; baseline (speedup 1.0000x reference)
import jax
import jax.numpy as jnp
from jax.experimental import pallas as pl


def kernel(sequence, segment_label, token_table, seg_table):
    raise NotImplementedError("write your pallas kernel here")


# speedup vs baseline: 6.3802x; 6.3802x over previous
import functools

import numpy as np
import jax
import jax.numpy as jnp
from jax import lax
from jax.experimental import pallas as pl
from jax.experimental.pallas import tpu as pltpu
from jax.experimental.pallas import tpu_sc as plsc

_VOCAB, _EMBED, _B, _S = 100000, 128, 1024, 200
_NC, _NS, _L = 2, 16, 16
_NW = _NC * _NS
_N = _B * _S
_RPW = _N // _NW
_CH = 128
_NCH = _RPW // _CH


def _pe_table():
    position = np.arange(_S, dtype=np.float32)[:, None]
    div_term = np.exp(
        np.arange(0, _EMBED, 2, dtype=np.float32) * (-np.log(10000.0) / _EMBED))
    pe = np.zeros((_S, _EMBED), dtype=np.float32)
    pe[:, 0::2] = np.sin(position * div_term)
    pe[:, 1::2] = np.cos(position * div_term)
    return pe


_PE = _pe_table()


def _comb_body(pe_ref, seg_ref, out_ref):
    out_ref[...] = seg_ref[...][:, None, :] + pe_ref[...][None, :, :]


def _build_comb(seg_table):
    out = pl.pallas_call(
        _comb_body,
        out_shape=jax.ShapeDtypeStruct((3, _S, _EMBED), jnp.float32),
    )(jnp.asarray(_PE), seg_table)
    return out.reshape(3 * _S, _EMBED)


_mesh = plsc.VectorSubcoreMesh(core_axis_name="c", subcore_axis_name="s")


@functools.partial(
    pl.kernel,
    out_type=jax.ShapeDtypeStruct((_N, _EMBED), jnp.float32),
    mesh=_mesh,
    scratch_types=[
        pltpu.VMEM((_RPW,), jnp.int32),
        pltpu.VMEM((_RPW,), jnp.int32),
        pltpu.VMEM((_CH, _EMBED), jnp.float32),
        pltpu.VMEM((_CH, _EMBED), jnp.float32),
        pltpu.SemaphoreType.DMA,
        pltpu.SemaphoreType.DMA,
    ],
)
def _emb(seq_hbm, seg_hbm, tok_hbm, comb_hbm, out_hbm,
         idx_v, ci_v, tok_b, comb_b, tsem, csem):
    wid = lax.axis_index("s") * _NC + lax.axis_index("c")
    base = wid * _RPW
    pltpu.sync_copy(seq_hbm.at[pl.ds(base, _RPW)], idx_v)
    pltpu.sync_copy(seg_hbm.at[pl.ds(base, _RPW)], ci_v)

    lane = lax.iota(jnp.int32, _L)

    @pl.loop(0, _RPW // _L)
    def _ci(i):
        off = pl.ds(i * _L, _L)
        pos = (lane + i * _L) % _S
        ci_v[off] = ci_v[off] * _S + pos

    @pl.loop(0, _NCH)
    def _chunk(k):
        rb = pl.multiple_of(k * _CH, _CH)
        tok_cp = pltpu.async_copy(tok_hbm.at[idx_v.at[pl.ds(rb, _CH)]], tok_b, tsem)
        comb_cp = pltpu.async_copy(comb_hbm.at[ci_v.at[pl.ds(rb, _CH)]], comb_b, csem)
        tok_cp.wait()
        comb_cp.wait()

        @pl.loop(0, _CH)
        def _row(r):
            for c in range(_EMBED // _L):
                sl = pl.ds(c * _L, _L)
                tok_b[r, sl] = tok_b[r, sl] + comb_b[r, sl]

        pltpu.sync_copy(tok_b, out_hbm.at[pl.ds(base + rb, _CH)])


def kernel(sequence, segment_label, token_table, seg_table):
    comb = _build_comb(seg_table)
    seq = sequence.reshape(-1).astype(jnp.int32)
    seg = segment_label.reshape(-1).astype(jnp.int32)
    out = _emb(seq, seg, token_table, comb)
    return out.reshape(_B, _S, _EMBED)


# speedup vs baseline: 7.8854x; 1.2359x over previous
import functools

import numpy as np
import jax
import jax.numpy as jnp
from jax import lax
from jax.experimental import pallas as pl
from jax.experimental.pallas import tpu as pltpu
from jax.experimental.pallas import tpu_sc as plsc

_VOCAB, _EMBED, _B, _S = 100000, 128, 1024, 200
_NC, _NS, _L = 2, 16, 16
_NW = _NC * _NS
_N = _B * _S
_RPW = _N // _NW
_CH = 128
_NCH = _RPW // _CH


def _pe_table():
    position = np.arange(_S, dtype=np.float32)[:, None]
    div_term = np.exp(
        np.arange(0, _EMBED, 2, dtype=np.float32) * (-np.log(10000.0) / _EMBED))
    pe = np.zeros((_S, _EMBED), dtype=np.float32)
    pe[:, 0::2] = np.sin(position * div_term)
    pe[:, 1::2] = np.cos(position * div_term)
    return pe


_PE = _pe_table()


def _comb_body(pe_ref, seg_ref, out_ref):
    out_ref[...] = seg_ref[...][:, None, :] + pe_ref[...][None, :, :]


def _build_comb(seg_table):
    out = pl.pallas_call(
        _comb_body,
        out_shape=jax.ShapeDtypeStruct((3, _S, _EMBED), jnp.float32),
    )(jnp.asarray(_PE), seg_table)
    return out.reshape(3 * _S, _EMBED)


_mesh = plsc.VectorSubcoreMesh(core_axis_name="c", subcore_axis_name="s")


@functools.partial(
    pl.kernel,
    out_type=jax.ShapeDtypeStruct((_N, _EMBED), jnp.float32),
    mesh=_mesh,
    scratch_types=[
        pltpu.VMEM((_RPW,), jnp.int32),
        pltpu.VMEM((_RPW,), jnp.int32),
        pltpu.VMEM((2, _CH, _EMBED), jnp.float32),
        pltpu.VMEM((2, _CH, _EMBED), jnp.float32),
        pltpu.SemaphoreType.DMA,
        pltpu.SemaphoreType.DMA,
        pltpu.SemaphoreType.DMA,
        pltpu.SemaphoreType.DMA,
    ],
)
def _emb(seq_hbm, seg_hbm, tok_hbm, comb_hbm, out_hbm,
         idx_v, ci_v, tok_b, comb_b, gsem0, gsem1, osem0, osem1):
    wid = lax.axis_index("s") * _NC + lax.axis_index("c")
    base = wid * _RPW
    pltpu.sync_copy(seq_hbm.at[pl.ds(base, _RPW)], idx_v)
    pltpu.sync_copy(seg_hbm.at[pl.ds(base, _RPW)], ci_v)

    gsem = (gsem0, gsem1)
    osem = (osem0, osem1)
    lane = lax.iota(jnp.int32, _L)

    @pl.loop(0, _RPW // _L)
    def _ci(i):
        off = pl.ds(i * _L, _L)
        pos = (lane + i * _L) % _S
        ci_v[off] = ci_v[off] * _S + pos

    def issue_gathers(k, b):
        rb = pl.multiple_of(k * _CH, _CH)
        pltpu.async_copy(tok_hbm.at[idx_v.at[pl.ds(rb, _CH)]], tok_b.at[b], gsem[b])
        pltpu.async_copy(comb_hbm.at[ci_v.at[pl.ds(rb, _CH)]], comb_b.at[b], gsem[b])

    def wait_gathers(k, b):
        rb = pl.multiple_of(k * _CH, _CH)
        pltpu.make_async_copy(tok_hbm.at[idx_v.at[pl.ds(rb, _CH)]], tok_b.at[b], gsem[b]).wait()
        pltpu.make_async_copy(comb_hbm.at[ci_v.at[pl.ds(rb, _CH)]], comb_b.at[b], gsem[b]).wait()

    def wait_out(b):
        pltpu.make_async_copy(tok_b.at[b], out_hbm.at[pl.ds(0, _CH)], osem[b]).wait()

    issue_gathers(0, 0)

    @pl.loop(0, _NCH // 2)
    def _pair(k2):
        for b in range(2):
            k = k2 * 2 + b

            @pl.when(k >= 1)
            def _():
                wait_out(1 - b)

            @pl.when(k + 1 < _NCH)
            def _():
                issue_gathers(k + 1, 1 - b)

            wait_gathers(k, b)

            @pl.loop(0, _CH)
            def _row(r):
                for c in range(_EMBED // _L):
                    sl = pl.ds(c * _L, _L)
                    tok_b[b, r, sl] = tok_b[b, r, sl] + comb_b[b, r, sl]

            rb = pl.multiple_of(k * _CH, _CH)
            pltpu.async_copy(tok_b.at[b], out_hbm.at[pl.ds(base + rb, _CH)], osem[b])

    wait_out((_NCH - 1) % 2)


def kernel(sequence, segment_label, token_table, seg_table):
    comb = _build_comb(seg_table)
    seq = sequence.reshape(-1).astype(jnp.int32)
    seg = segment_label.reshape(-1).astype(jnp.int32)
    out = _emb(seq, seg, token_table, comb)
    return out.reshape(_B, _S, _EMBED)
